# Initial kernel scaffold; baseline (speedup 1.0000x reference)
import jax
import jax.numpy as jnp
from jax.experimental import pallas as pl


def kernel(concept_ids, embeddings, W1, b1, W2, b2):
    raise NotImplementedError("write your pallas kernel here")


# speedup vs baseline: 1.2512x; 1.2512x over previous
import functools

import jax
import jax.numpy as jnp
from jax import lax
from jax.experimental import pallas as pl
from jax.experimental.pallas import tpu as pltpu
from jax.experimental.pallas import tpu_sc as plsc

_VOCAB = 1000
_D = 64
_HIDDEN = 64
_BATCH = 16384

_INFO = plsc.get_sparse_core_info()
_NC = _INFO.num_cores
_NS = _INFO.num_subcores
_L = _INFO.num_lanes
_NW = _NC * _NS

_BPW = _BATCH // _NW
_CHUNK = 128
_NCHUNK = (2 * _BPW) // _CHUNK


def _compose_kernel(ids_hbm, table_hbm, out_hbm, idx_v, rows_v, acc_v, sem):
    wid = lax.axis_index("s") * _NC + lax.axis_index("c")
    base = wid * _BPW

    pltpu.sync_copy(ids_hbm.at[pl.ds(wid * _NCHUNK, _NCHUNK)], idx_v)

    copies = []
    for k in range(_NCHUNK):
        copies.append(
            pltpu.async_copy(table_hbm.at[idx_v.at[k]], rows_v.at[k], sem))
    for c in copies:
        c.wait()

    rows_per_chunk = _CHUNK // 2

    for k in range(_NCHUNK):
        def body(i, _, k=k):
            for j in range(_D // _L):
                sl = pl.ds(j * _L, _L)
                acc_v[k * rows_per_chunk + i, sl] = (
                    rows_v[k, 2 * i, sl] + rows_v[k, 2 * i + 1, sl])
            return 0
        lax.fori_loop(0, rows_per_chunk, body, 0)

    pltpu.sync_copy(acc_v, out_hbm.at[pl.ds(base, _BPW)])


_compose = functools.partial(
    pl.kernel,
    mesh=plsc.VectorSubcoreMesh(core_axis_name="c", subcore_axis_name="s"),
    out_type=jax.ShapeDtypeStruct((_BATCH, _D), jnp.float32),
    scratch_types=[
        pltpu.VMEM((_NCHUNK, _CHUNK), jnp.int32),
        pltpu.VMEM((_NCHUNK, _CHUNK, _D), jnp.float32),
        pltpu.VMEM((_BPW, _D), jnp.float32),
        pltpu.SemaphoreType.DMA,
    ],
    compiler_params=pltpu.CompilerParams(use_tc_tiling_on_sc=False),
)(_compose_kernel)


_BLK = 512


def _mlp_kernel(x_ref, w1_ref, b1_ref, w2_ref, b2_ref, o_ref):
    x = x_ref[...]
    h = jnp.maximum(
        jnp.dot(x, w1_ref[...], preferred_element_type=jnp.float32)
        + b1_ref[...], 0.0)
    logits = (jnp.dot(h, w2_ref[...], preferred_element_type=jnp.float32)
              + b2_ref[...])
    m = jnp.max(logits, axis=1, keepdims=True)
    s = logits - m
    lse = jnp.log(jnp.sum(jnp.exp(s), axis=1, keepdims=True))
    o_ref[...] = s - lse


_mlp = pl.pallas_call(
    _mlp_kernel,
    grid=(_BATCH // _BLK,),
    in_specs=[
        pl.BlockSpec((_BLK, _D), lambda i: (i, 0)),
        pl.BlockSpec((_D, _HIDDEN), lambda i: (0, 0)),
        pl.BlockSpec((1, _HIDDEN), lambda i: (0, 0)),
        pl.BlockSpec((_HIDDEN, _VOCAB), lambda i: (0, 0)),
        pl.BlockSpec((1, _VOCAB), lambda i: (0, 0)),
    ],
    out_specs=pl.BlockSpec((_BLK, _VOCAB), lambda i: (i, 0)),
    out_shape=jax.ShapeDtypeStruct((_BATCH, _VOCAB), jnp.float32),
    compiler_params=pltpu.CompilerParams(
        dimension_semantics=("parallel",)),
)


def kernel(concept_ids, embeddings, W1, b1, W2, b2):
    ids2d = concept_ids.reshape(_NW * _NCHUNK, _CHUNK)
    composed = _compose(ids2d, embeddings)
    return _mlp(composed, W1, b1.reshape(1, _HIDDEN), W2,
                b2.reshape(1, _VOCAB))


# speedup vs baseline: 1.9198x; 1.5343x over previous
import functools

import jax
import jax.numpy as jnp
from jax import lax
from jax.experimental import pallas as pl
from jax.experimental.pallas import tpu as pltpu
from jax.experimental.pallas import tpu_sc as plsc

_VOCAB = 1000
_D = 64
_HIDDEN = 64
_BATCH = 16384

_INFO = plsc.get_sparse_core_info()
_NC = _INFO.num_cores
_NS = _INFO.num_subcores
_L = _INFO.num_lanes
_NW = _NC * _NS

_BPW = _BATCH // _NW
_CHUNK = 128
_NCHUNK = (2 * _BPW) // _CHUNK


def _compose_kernel(ids_hbm, table_hbm, out_hbm, idx_v, rows_v, acc_v, sem):
    wid = lax.axis_index("s") * _NC + lax.axis_index("c")
    base = wid * _BPW

    pltpu.sync_copy(ids_hbm.at[pl.ds(wid * _NCHUNK, _NCHUNK)], idx_v)

    copies = []
    for k in range(_NCHUNK):
        copies.append(
            pltpu.async_copy(table_hbm.at[idx_v.at[k]], rows_v.at[k], sem))
    for c in copies:
        c.wait()

    rows_per_chunk = _CHUNK // 2

    for k in range(_NCHUNK):
        def body(i, _, k=k):
            for j in range(_D // _L):
                sl = pl.ds(j * _L, _L)
                acc_v[k * rows_per_chunk + i, sl] = (
                    rows_v[k, 2 * i, sl] + rows_v[k, 2 * i + 1, sl])
            return 0
        lax.fori_loop(0, rows_per_chunk, body, 0)

    pltpu.sync_copy(acc_v, out_hbm.at[pl.ds(base, _BPW)])


_compose = functools.partial(
    pl.kernel,
    mesh=plsc.VectorSubcoreMesh(core_axis_name="c", subcore_axis_name="s"),
    out_type=jax.ShapeDtypeStruct((_BATCH, _D), jnp.float32),
    scratch_types=[
        pltpu.VMEM((_NCHUNK, _CHUNK), jnp.int32),
        pltpu.VMEM((_NCHUNK, _CHUNK, _D), jnp.float32),
        pltpu.VMEM((_BPW, _D), jnp.float32),
        pltpu.SemaphoreType.DMA,
    ],
    compiler_params=pltpu.CompilerParams(use_tc_tiling_on_sc=False),
)(_compose_kernel)


_BLK = 512


def _mlp_t_kernel(x_ref, w1t_ref, b1t_ref, w2t_ref, b2t_ref, o_ref):
    x = x_ref[...]
    ht = lax.dot_general(w1t_ref[...], x, (((1,), (1,)), ((), ())),
                         preferred_element_type=jnp.float32)
    ht = jnp.maximum(ht + b1t_ref[...], 0.0)
    lt = lax.dot_general(w2t_ref[...], ht, (((1,), (0,)), ((), ())),
                         preferred_element_type=jnp.float32)
    lt = lt + b2t_ref[...]
    m = jnp.max(lt, axis=0, keepdims=True)
    s = lt - m
    lse = jnp.log(jnp.sum(jnp.exp(s), axis=0, keepdims=True))
    o_ref[...] = s - lse


_mlp_t = pl.pallas_call(
    _mlp_t_kernel,
    grid=(_BATCH // _BLK,),
    in_specs=[
        pl.BlockSpec((_BLK, _D), lambda i: (i, 0)),
        pl.BlockSpec((_HIDDEN, _D), lambda i: (0, 0)),
        pl.BlockSpec((_HIDDEN, 1), lambda i: (0, 0)),
        pl.BlockSpec((_VOCAB, _HIDDEN), lambda i: (0, 0)),
        pl.BlockSpec((_VOCAB, 1), lambda i: (0, 0)),
    ],
    out_specs=pl.BlockSpec((_VOCAB, _BLK), lambda i: (0, i)),
    out_shape=jax.ShapeDtypeStruct((_VOCAB, _BATCH), jnp.float32),
    compiler_params=pltpu.CompilerParams(
        dimension_semantics=("parallel",)),
)


def kernel(concept_ids, embeddings, W1, b1, W2, b2):
    ids2d = concept_ids.reshape(_NW * _NCHUNK, _CHUNK)
    composed = _compose(ids2d, embeddings)
    out_t = _mlp_t(composed, W1.T, b1.reshape(_HIDDEN, 1), W2.T,
                   b2.reshape(_VOCAB, 1))
    return out_t.T


# speedup vs baseline: 2.3903x; 1.2451x over previous
import functools

import jax
import jax.numpy as jnp
from jax import lax
from jax.experimental import pallas as pl
from jax.experimental.pallas import tpu as pltpu
from jax.experimental.pallas import tpu_sc as plsc

_VOCAB = 1000
_D = 64
_HIDDEN = 64
_BATCH = 16384

_INFO = plsc.get_sparse_core_info()
_NC = _INFO.num_cores
_NS = _INFO.num_subcores
_L = _INFO.num_lanes
_NW = _NC * _NS

_BPW = _BATCH // _NW
_CHUNK = 128
_NCHUNK = (2 * _BPW) // _CHUNK


def _compose_kernel(ids_hbm, table_hbm, out_hbm, idx_v, rows_v, acc_v, sem):
    wid = lax.axis_index("s") * _NC + lax.axis_index("c")
    base = wid * _BPW

    pltpu.sync_copy(ids_hbm.at[pl.ds(wid * _NCHUNK, _NCHUNK)], idx_v)

    copies = []
    for k in range(_NCHUNK):
        copies.append(
            pltpu.async_copy(table_hbm.at[idx_v.at[k]], rows_v.at[k], sem))
    for c in copies:
        c.wait()

    for c in range(_NCHUNK // 2):
        def body(l, _, c=c):
            for j in range(_D // _L):
                sl = pl.ds(j * _L, _L)
                acc_v[c * _CHUNK + l, sl] = (
                    rows_v[2 * c, l, sl] + rows_v[2 * c + 1, l, sl])
            return 0
        lax.fori_loop(0, _CHUNK, body, 0)

    pltpu.sync_copy(acc_v, out_hbm.at[pl.ds(base, _BPW)])


_compose = functools.partial(
    pl.kernel,
    mesh=plsc.VectorSubcoreMesh(core_axis_name="c", subcore_axis_name="s"),
    out_type=jax.ShapeDtypeStruct((_BATCH, _D), jnp.float32),
    scratch_types=[
        pltpu.VMEM((_NCHUNK, _CHUNK), jnp.int32),
        pltpu.VMEM((_NCHUNK, _CHUNK, _D), jnp.float32),
        pltpu.VMEM((_BPW, _D), jnp.float32),
        pltpu.SemaphoreType.DMA,
    ],
    compiler_params=pltpu.CompilerParams(use_tc_tiling_on_sc=False),
)(_compose_kernel)


_BLK = 512


def _mlp_t_kernel(x_ref, w1t_ref, b1t_ref, w2t_ref, b2t_ref, o_ref):
    x = x_ref[...]
    ht = lax.dot_general(w1t_ref[...], x, (((1,), (1,)), ((), ())),
                         preferred_element_type=jnp.float32)
    ht = jnp.maximum(ht + b1t_ref[...], 0.0)
    lt = lax.dot_general(w2t_ref[...], ht, (((1,), (0,)), ((), ())),
                         preferred_element_type=jnp.float32)
    lt = lt + b2t_ref[...]
    m = jnp.max(lt, axis=0, keepdims=True)
    s = lt - m
    lse = jnp.log(jnp.sum(jnp.exp(s), axis=0, keepdims=True))
    o_ref[...] = s - lse


_mlp_t = pl.pallas_call(
    _mlp_t_kernel,
    grid=(_BATCH // _BLK,),
    in_specs=[
        pl.BlockSpec((_BLK, _D), lambda i: (i, 0)),
        pl.BlockSpec((_HIDDEN, _D), lambda i: (0, 0)),
        pl.BlockSpec((_HIDDEN, 1), lambda i: (0, 0)),
        pl.BlockSpec((_VOCAB, _HIDDEN), lambda i: (0, 0)),
        pl.BlockSpec((_VOCAB, 1), lambda i: (0, 0)),
    ],
    out_specs=pl.BlockSpec((_VOCAB, _BLK), lambda i: (0, i)),
    out_shape=jax.ShapeDtypeStruct((_VOCAB, _BATCH), jnp.float32),
    compiler_params=pltpu.CompilerParams(
        dimension_semantics=("parallel",)),
)


def kernel(concept_ids, embeddings, W1, b1, W2, b2):
    ids2d = (concept_ids.reshape(_BATCH // _CHUNK, _CHUNK, 2)
             .transpose(0, 2, 1).reshape(_NW * _NCHUNK, _CHUNK))
    composed = _compose(ids2d, embeddings)
    out_t = _mlp_t(composed, W1.T, b1.reshape(_HIDDEN, 1), W2.T,
                   b2.reshape(_VOCAB, 1))
    return out_t.T


# speedup vs baseline: 2.6146x; 1.0938x over previous
import functools

import jax
import jax.numpy as jnp
from jax import lax
from jax.experimental import pallas as pl
from jax.experimental.pallas import tpu as pltpu
from jax.experimental.pallas import tpu_sc as plsc

_VOCAB = 1000
_D = 64
_HIDDEN = 64
_BATCH = 16384

_INFO = plsc.get_sparse_core_info()
_NC = _INFO.num_cores
_NS = _INFO.num_subcores
_L = _INFO.num_lanes
_NW = _NC * _NS

_BPW = _BATCH // _NW
_CHUNK = 128
_NCHUNK = (2 * _BPW) // _CHUNK


def _compose_kernel(ids_hbm, table_hbm, out_hbm, idx_v, rows_v, acc_v, sem):
    wid = lax.axis_index("s") * _NC + lax.axis_index("c")
    base = wid * _BPW

    pltpu.sync_copy(ids_hbm.at[pl.ds(wid * _NCHUNK, _NCHUNK)], idx_v)

    copies = []
    for k in range(_NCHUNK):
        copies.append(
            pltpu.async_copy(table_hbm.at[idx_v.at[k]], rows_v.at[k], sem))
    for c in copies:
        c.wait()

    for c in range(_NCHUNK // 2):
        def body(l, _, c=c):
            for j in range(_D // _L):
                sl = pl.ds(j * _L, _L)
                acc_v[c * _CHUNK + l, sl] = (
                    rows_v[2 * c, l, sl] + rows_v[2 * c + 1, l, sl])
            return 0
        lax.fori_loop(0, _CHUNK, body, 0)

    pltpu.sync_copy(acc_v, out_hbm.at[pl.ds(base, _BPW), pl.ds(0, _D)])


_compose = functools.partial(
    pl.kernel,
    mesh=plsc.VectorSubcoreMesh(core_axis_name="c", subcore_axis_name="s"),
    out_type=jax.ShapeDtypeStruct((_BATCH, 2 * _D), jnp.float32),
    scratch_types=[
        pltpu.VMEM((_NCHUNK, _CHUNK), jnp.int32),
        pltpu.VMEM((_NCHUNK, _CHUNK, _D), jnp.float32),
        pltpu.VMEM((_BPW, _D), jnp.float32),
        pltpu.SemaphoreType.DMA,
    ],
    compiler_params=pltpu.CompilerParams(use_tc_tiling_on_sc=False),
)(_compose_kernel)


_BLK = 512


def _mlp_t_kernel(x_ref, w1t_ref, b1t_ref, w2t_ref, b2t_ref, o_ref):
    x = x_ref[:, : _D]
    ht = lax.dot_general(w1t_ref[...], x, (((1,), (1,)), ((), ())),
                         preferred_element_type=jnp.float32)
    ht = jnp.maximum(ht + b1t_ref[...], 0.0)
    lt = lax.dot_general(w2t_ref[...], ht, (((1,), (0,)), ((), ())),
                         preferred_element_type=jnp.float32)
    lt = lt + b2t_ref[...]
    m = jnp.max(lt, axis=0, keepdims=True)
    s = lt - m
    lse = jnp.log(jnp.sum(jnp.exp(s), axis=0, keepdims=True))
    o_ref[...] = s - lse


_mlp_t = pl.pallas_call(
    _mlp_t_kernel,
    grid=(_BATCH // _BLK,),
    in_specs=[
        pl.BlockSpec((_BLK, 2 * _D), lambda i: (i, 0)),
        pl.BlockSpec((_HIDDEN, _D), lambda i: (0, 0)),
        pl.BlockSpec((_HIDDEN, 1), lambda i: (0, 0)),
        pl.BlockSpec((_VOCAB, _HIDDEN), lambda i: (0, 0)),
        pl.BlockSpec((_VOCAB, 1), lambda i: (0, 0)),
    ],
    out_specs=pl.BlockSpec((_VOCAB, _BLK), lambda i: (0, i)),
    out_shape=jax.ShapeDtypeStruct((_VOCAB, _BATCH), jnp.float32),
    compiler_params=pltpu.CompilerParams(
        dimension_semantics=("parallel",)),
)


def kernel(concept_ids, embeddings, W1, b1, W2, b2):
    ids2d = (concept_ids.reshape(_BATCH // _CHUNK, _CHUNK, 2)
             .transpose(0, 2, 1).reshape(_NW * _NCHUNK, _CHUNK))
    composed = _compose(ids2d, embeddings)
    out_t = _mlp_t(composed, W1.T, b1.reshape(_HIDDEN, 1), W2.T,
                   b2.reshape(_VOCAB, 1))
    return out_t.T


# speedup vs baseline: 2.8985x; 1.1086x over previous
import functools

import jax
import jax.numpy as jnp
from jax import lax
from jax.experimental import pallas as pl
from jax.experimental.pallas import tpu as pltpu
from jax.experimental.pallas import tpu_sc as plsc

_VOCAB = 1000
_D = 64
_HIDDEN = 64
_BATCH = 16384

_INFO = plsc.get_sparse_core_info()
_NC = _INFO.num_cores
_NS = _INFO.num_subcores
_L = _INFO.num_lanes
_NW = _NC * _NS

_BPW = _BATCH // _NW
_CHUNK = 128
_NCHUNK = (2 * _BPW) // _CHUNK


def _compose_kernel(ids_hbm, table_hbm, out_hbm, idx_v, rows_v, acc_v, sem):
    wid = lax.axis_index("s") * _NC + lax.axis_index("c")
    base = wid * _BPW

    pltpu.sync_copy(ids_hbm.at[pl.ds(wid * _NCHUNK, _NCHUNK)], idx_v)

    copies = []
    for k in range(_NCHUNK):
        copies.append(
            pltpu.async_copy(table_hbm.at[idx_v.at[k]], rows_v.at[k], sem))
    for c in copies:
        c.wait()

    for c in range(_NCHUNK // 2):
        def body(l, _, c=c):
            for j in range(_D // _L):
                sl = pl.ds(j * _L, _L)
                acc_v[c * _CHUNK + l, sl] = (
                    rows_v[2 * c, l, sl] + rows_v[2 * c + 1, l, sl])
            return 0
        lax.fori_loop(0, _CHUNK, body, 0)

    pltpu.sync_copy(acc_v, out_hbm.at[pl.ds(base, _BPW), pl.ds(0, _D)])


_compose = functools.partial(
    pl.kernel,
    mesh=plsc.VectorSubcoreMesh(core_axis_name="c", subcore_axis_name="s"),
    out_type=jax.ShapeDtypeStruct((_BATCH, 2 * _D), jnp.float32),
    scratch_types=[
        pltpu.VMEM((_NCHUNK, _CHUNK), jnp.int32),
        pltpu.VMEM((_NCHUNK, _CHUNK, _D), jnp.float32),
        pltpu.VMEM((_BPW, _D), jnp.float32),
        pltpu.SemaphoreType.DMA,
    ],
    compiler_params=pltpu.CompilerParams(use_tc_tiling_on_sc=False),
)(_compose_kernel)


_BLK = 1024


def _mlp_t_kernel(x_ref, w1t_ref, b1t_ref, w2t_ref, b2t_ref, o_ref):
    x = x_ref[:, : _D]
    ht = lax.dot_general(w1t_ref[...], x, (((1,), (1,)), ((), ())),
                         preferred_element_type=jnp.float32)
    ht = jnp.maximum(ht + b1t_ref[...], 0.0)
    lt = lax.dot_general(w2t_ref[...], ht, (((1,), (0,)), ((), ())),
                         preferred_element_type=jnp.float32)
    lt = lt + b2t_ref[...]
    m = jnp.max(lt, axis=0, keepdims=True)
    s = lt - m
    lse = jnp.log(jnp.sum(jnp.exp(s), axis=0, keepdims=True))
    o_ref[...] = s - lse


_mlp_t = pl.pallas_call(
    _mlp_t_kernel,
    grid=(_BATCH // _BLK,),
    in_specs=[
        pl.BlockSpec((_BLK, 2 * _D), lambda i: (i, 0)),
        pl.BlockSpec((_HIDDEN, _D), lambda i: (0, 0)),
        pl.BlockSpec((_HIDDEN, 1), lambda i: (0, 0)),
        pl.BlockSpec((_VOCAB, _HIDDEN), lambda i: (0, 0)),
        pl.BlockSpec((_VOCAB, 1), lambda i: (0, 0)),
    ],
    out_specs=pl.BlockSpec((_VOCAB, _BLK), lambda i: (0, i)),
    out_shape=jax.ShapeDtypeStruct((_VOCAB, _BATCH), jnp.float32),
    compiler_params=pltpu.CompilerParams(
        dimension_semantics=("parallel",)),
)


def kernel(concept_ids, embeddings, W1, b1, W2, b2):
    ids2d = (concept_ids.reshape(_BATCH // _CHUNK, _CHUNK, 2)
             .transpose(0, 2, 1).reshape(_NW * _NCHUNK, _CHUNK))
    composed = _compose(ids2d, embeddings)
    out_t = _mlp_t(composed, W1.T, b1.reshape(_HIDDEN, 1), W2.T,
                   b2.reshape(_VOCAB, 1))
    return out_t.T


# speedup vs baseline: 3.0080x; 1.0378x over previous
import functools

import jax
import jax.numpy as jnp
from jax import lax
from jax.experimental import pallas as pl
from jax.experimental.pallas import tpu as pltpu
from jax.experimental.pallas import tpu_sc as plsc

_VOCAB = 1000
_D = 64
_HIDDEN = 64
_BATCH = 16384

_INFO = plsc.get_sparse_core_info()
_NC = _INFO.num_cores
_NS = _INFO.num_subcores
_L = _INFO.num_lanes
_NW = _NC * _NS

_BPW = _BATCH // _NW
_CHUNK = 128
_NCHUNK = (2 * _BPW) // _CHUNK


def _compose_kernel(ids_hbm, table_hbm, out_hbm, idx_v, rows_v, acc_v, sem):
    wid = lax.axis_index("s") * _NC + lax.axis_index("c")
    base = wid * _BPW

    pltpu.sync_copy(ids_hbm.at[pl.ds(wid * _NCHUNK, _NCHUNK)], idx_v)

    copies = []
    for k in range(_NCHUNK):
        copies.append(
            pltpu.async_copy(table_hbm.at[idx_v.at[k]], rows_v.at[k], sem))
    for c in copies:
        c.wait()

    for c in range(_NCHUNK // 2):
        def body(l, _, c=c):
            for j in range(_D // _L):
                sl = pl.ds(j * _L, _L)
                acc_v[c * _CHUNK + l, sl] = (
                    rows_v[2 * c, l, sl] + rows_v[2 * c + 1, l, sl])
            return 0
        lax.fori_loop(0, _CHUNK, body, 0)

    pltpu.sync_copy(acc_v, out_hbm.at[pl.ds(base, _BPW), pl.ds(0, _D)])


_compose = functools.partial(
    pl.kernel,
    mesh=plsc.VectorSubcoreMesh(core_axis_name="c", subcore_axis_name="s"),
    out_type=jax.ShapeDtypeStruct((_BATCH, 2 * _D), jnp.float32),
    scratch_types=[
        pltpu.VMEM((_NCHUNK, _CHUNK), jnp.int32),
        pltpu.VMEM((_NCHUNK, _CHUNK, _D), jnp.float32),
        pltpu.VMEM((_BPW, _D), jnp.float32),
        pltpu.SemaphoreType.DMA,
    ],
    compiler_params=pltpu.CompilerParams(use_tc_tiling_on_sc=False),
)(_compose_kernel)


_BLK = 2048


def _mlp_t_kernel(x_ref, w1t_ref, b1t_ref, w2t_ref, b2t_ref, o_ref):
    x = x_ref[:, : _D]
    ht = lax.dot_general(w1t_ref[...], x, (((1,), (1,)), ((), ())),
                         preferred_element_type=jnp.float32)
    ht = jnp.maximum(ht + b1t_ref[...], 0.0)
    lt = lax.dot_general(w2t_ref[...], ht, (((1,), (0,)), ((), ())),
                         preferred_element_type=jnp.float32)
    lt = lt + b2t_ref[...]
    m = jnp.max(lt, axis=0, keepdims=True)
    s = lt - m
    lse = jnp.log(jnp.sum(jnp.exp(s), axis=0, keepdims=True))
    o_ref[...] = s - lse


_mlp_t = pl.pallas_call(
    _mlp_t_kernel,
    grid=(_BATCH // _BLK,),
    in_specs=[
        pl.BlockSpec((_BLK, 2 * _D), lambda i: (i, 0)),
        pl.BlockSpec((_HIDDEN, _D), lambda i: (0, 0)),
        pl.BlockSpec((_HIDDEN, 1), lambda i: (0, 0)),
        pl.BlockSpec((_VOCAB, _HIDDEN), lambda i: (0, 0)),
        pl.BlockSpec((_VOCAB, 1), lambda i: (0, 0)),
    ],
    out_specs=pl.BlockSpec((_VOCAB, _BLK), lambda i: (0, i)),
    out_shape=jax.ShapeDtypeStruct((_VOCAB, _BATCH), jnp.float32),
    compiler_params=pltpu.CompilerParams(
        dimension_semantics=("parallel",)),
)


def kernel(concept_ids, embeddings, W1, b1, W2, b2):
    ids2d = (concept_ids.reshape(_BATCH // _CHUNK, _CHUNK, 2)
             .transpose(0, 2, 1).reshape(_NW * _NCHUNK, _CHUNK))
    composed = _compose(ids2d, embeddings)
    out_t = _mlp_t(composed, W1.T, b1.reshape(_HIDDEN, 1), W2.T,
                   b2.reshape(_VOCAB, 1))
    return out_t.T
